# baseline (speedup 1.0000x reference)
import functools

import jax
import jax.numpy as jnp
from jax import lax
from jax.experimental import pallas as pl
from jax.experimental.pallas import tpu as pltpu
from jax.experimental.pallas import tpu_sc as plsc

_TILE = 4096
_CHUNK = 64


def _logits_kernel(e_ref, w1_ref, b1_ref, w2_ref, lo_ref, m_ref_o, z_ref_o,
                   m_ref, z_ref, *, tile, batch):
    i = pl.program_id(0)

    @pl.when(i == 0)
    def _init():
        m_ref[0] = -jnp.inf
        z_ref[0] = 0.0

    e = e_ref[...]
    h = jax.lax.dot_general(
        e, w1_ref[...], (((1,), (1,)), ((), ())),
        preferred_element_type=jnp.float32)
    h = jnp.maximum(h + b1_ref[...], 0.0)
    logit = jax.lax.dot_general(
        h, w2_ref[...], (((1,), (1,)), ((), ())),
        preferred_element_type=jnp.float32)
    lo_ref[...] = logit

    m_old = m_ref[0]
    m_new = jnp.maximum(m_old, jnp.max(logit))
    alpha = jnp.exp(m_old - m_new)
    z_ref[0] = z_ref[0] * alpha + jnp.sum(jnp.exp(logit - m_new))
    m_ref[0] = m_new

    @pl.when(i == pl.num_programs(0) - 1)
    def _finish():
        m_ref_o[...] = jnp.full((1, 16), m_ref[0], jnp.float32)
        z_ref_o[...] = jnp.full((1, 16), z_ref[0], jnp.float32)


def _seg_sum_sc(e_hbm, lo_hbm, m_hbm, en_hbm, part_hbm,
                e_buf, l_buf, s_buf, acc_ref, m_buf, en_buf,
                *, total, embed_dim, chunk):
    wid = lax.axis_index("s") * 2 + lax.axis_index("c")
    seg = wid // 2
    part = wid % 2

    pltpu.sync_copy(m_hbm, m_buf)
    pltpu.sync_copy(en_hbm, en_buf.at[pl.ds(0, 16)])
    mv = m_buf[...]

    for k in range(embed_dim // 16):
        acc_ref[pl.ds(k * 16, 16)] = jnp.zeros((16,), jnp.float32)

    en_seg = jnp.minimum(en_buf[pl.ds(seg, 16)][0], total)
    st_seg = jnp.where(
        seg == 0, 0,
        jnp.minimum(en_buf[pl.ds(jnp.maximum(seg - 1, 0), 16)][0], total))
    half = (en_seg - st_seg + 1) // 2
    my_start = st_seg + part * half
    my_end = jnp.minimum(en_seg, my_start + half)
    g0 = my_start // chunk
    nslice = embed_dim // 16

    def chunk_body(ci, _):
        base = (g0 + ci) * chunk

        @pl.when(base < my_end)
        def _():
            pltpu.sync_copy(e_hbm.at[pl.ds(base, chunk)], e_buf)
            pltpu.sync_copy(lo_hbm.at[pl.ds(base, chunk)], l_buf)
            for k4 in range(chunk // 16):
                s_buf[pl.ds(k4 * 16, 16)] = jnp.exp(
                    l_buf[pl.ds(k4 * 16, 16)] - mv)

            def tok_body(ti, _2):
                t = base + ti
                valid = jnp.logical_and(t >= my_start, t < my_end)
                w_eff = jnp.where(valid, s_buf[pl.ds(ti, 16)][0], 0.0)
                for k in range(nslice):
                    ev = e_buf[ti, pl.ds(k * 16, 16)]
                    plsc.addupdate(acc_ref.at[pl.ds(k * 16, 16)], ev * w_eff)
                return 0

            lax.fori_loop(0, chunk, tok_body, 0)
        return 0

    lax.fori_loop(0, total // chunk // 16 + 1, chunk_body, 0)
    pltpu.sync_copy(acc_ref, part_hbm.at[wid])


def _combine_kernel(p_ref, z_ref, out_ref):
    out_ref[...] = (p_ref[:, 0, :] + p_ref[:, 1, :]) / z_ref[0:1, 0:1]


def kernel(embeddings, lengths, W1, b1, W2, b2):
    total, embed_dim = embeddings.shape
    batch = lengths.shape[0]
    tile = _TILE
    num_tiles = total // tile

    logits, m_arr, z_arr = pl.pallas_call(
        functools.partial(_logits_kernel, tile=tile, batch=batch),
        grid=(num_tiles,),
        in_specs=[
            pl.BlockSpec((tile, embed_dim), lambda i: (i, 0)),
            pl.BlockSpec((embed_dim, embed_dim), lambda i: (0, 0)),
            pl.BlockSpec((1, embed_dim), lambda i: (0, 0)),
            pl.BlockSpec((1, embed_dim), lambda i: (0, 0)),
        ],
        out_specs=[
            pl.BlockSpec((tile, 1), lambda i: (i, 0)),
            pl.BlockSpec((1, 16), lambda i: (0, 0)),
            pl.BlockSpec((1, 16), lambda i: (0, 0)),
        ],
        out_shape=[
            jax.ShapeDtypeStruct((total, 1), jnp.float32),
            jax.ShapeDtypeStruct((1, 16), jnp.float32),
            jax.ShapeDtypeStruct((1, 16), jnp.float32),
        ],
        scratch_shapes=[
            pltpu.SMEM((1,), jnp.float32),
            pltpu.SMEM((1,), jnp.float32),
        ],
        compiler_params=pltpu.CompilerParams(
            dimension_semantics=("arbitrary",),
        ),
    )(embeddings, W1, b1.reshape(1, embed_dim), W2)

    ends = jnp.cumsum(lengths.astype(jnp.int32))

    mesh = plsc.VectorSubcoreMesh(core_axis_name="c", subcore_axis_name="s")
    sc_fn = functools.partial(
        _seg_sum_sc, total=total, embed_dim=embed_dim, chunk=_CHUNK)
    partials = pl.kernel(
        sc_fn,
        mesh=mesh,
        out_type=jax.ShapeDtypeStruct((2 * batch, embed_dim), jnp.float32),
        scratch_types=[
            pltpu.VMEM((_CHUNK, embed_dim), jnp.float32),
            pltpu.VMEM((_CHUNK,), jnp.float32),
            pltpu.VMEM((_CHUNK + 16,), jnp.float32),
            pltpu.VMEM((embed_dim,), jnp.float32),
            pltpu.VMEM((16,), jnp.float32),
            pltpu.VMEM((32,), jnp.int32),
        ],
    )(embeddings, logits.reshape(total), m_arr.reshape(16), ends)

    out = pl.pallas_call(
        _combine_kernel,
        in_specs=[
            pl.BlockSpec((batch, 2, embed_dim), lambda: (0, 0, 0)),
            pl.BlockSpec((1, 16), lambda: (0, 0)),
        ],
        out_specs=pl.BlockSpec((batch, embed_dim), lambda: (0, 0)),
        out_shape=jax.ShapeDtypeStruct((batch, embed_dim), jnp.float32),
    )(partials.reshape(batch, 2, embed_dim), z_arr)
    return out


# speedup vs baseline: 10.9020x; 10.9020x over previous
import functools

import jax
import jax.numpy as jnp
from jax.experimental import pallas as pl
from jax.experimental.pallas import tpu as pltpu

_TILE = 4096


def _dwa_kernel(e_ref, w1_ref, b1_ref, w2_ref, st_ref, en_ref, out_ref,
                acc_ref, m_ref, z_ref, *, tile, batch):
    i = pl.program_id(0)

    @pl.when(i == 0)
    def _init():
        m_ref[0] = -jnp.inf
        z_ref[0] = 0.0
        acc_ref[...] = jnp.zeros_like(acc_ref)

    e = e_ref[...]
    h = jax.lax.dot_general(
        e, w1_ref[...], (((1,), (1,)), ((), ())),
        preferred_element_type=jnp.float32)
    h = jnp.maximum(h + b1_ref[...], 0.0)
    logit = jax.lax.dot_general(
        h, w2_ref[...], (((1,), (1,)), ((), ())),
        preferred_element_type=jnp.float32)

    m_old = m_ref[0]
    m_new = jnp.maximum(m_old, jnp.max(logit))
    alpha = jnp.exp(m_old - m_new)
    s = jnp.exp(logit - m_new)
    z_ref[0] = z_ref[0] * alpha + jnp.sum(s)

    st = st_ref[...]
    en = en_ref[...]
    rows = jax.lax.broadcasted_iota(jnp.int32, (tile, batch), 0) + i * tile
    mask = jnp.logical_and(rows >= st, rows < en)
    masked = jnp.where(mask, s, 0.0)
    contrib = jax.lax.dot_general(
        masked, e, (((0,), (0,)), ((), ())),
        preferred_element_type=jnp.float32)
    acc_ref[...] = acc_ref[...] * alpha + contrib
    m_ref[0] = m_new

    @pl.when(i == pl.num_programs(0) - 1)
    def _finish():
        out_ref[...] = acc_ref[...] / z_ref[0]


def kernel(embeddings, lengths, W1, b1, W2, b2):
    total, embed_dim = embeddings.shape
    batch = lengths.shape[0]
    tile = _TILE
    num_tiles = total // tile

    out_call = pl.pallas_call(
        functools.partial(_dwa_kernel, tile=tile, batch=batch),
        grid=(num_tiles,),
        in_specs=[
            pl.BlockSpec((tile, embed_dim), lambda i: (i, 0)),
            pl.BlockSpec((embed_dim, embed_dim), lambda i: (0, 0)),
            pl.BlockSpec((1, embed_dim), lambda i: (0, 0)),
            pl.BlockSpec((1, embed_dim), lambda i: (0, 0)),
            pl.BlockSpec((1, batch), lambda i: (0, 0)),
            pl.BlockSpec((1, batch), lambda i: (0, 0)),
        ],
        out_specs=pl.BlockSpec((batch, embed_dim), lambda i: (0, 0)),
        out_shape=jax.ShapeDtypeStruct((batch, embed_dim), jnp.float32),
        scratch_shapes=[
            pltpu.VMEM((batch, embed_dim), jnp.float32),
            pltpu.SMEM((1,), jnp.float32),
            pltpu.SMEM((1,), jnp.float32),
        ],
        compiler_params=pltpu.CompilerParams(
            dimension_semantics=("arbitrary",),
        ),
    )
    ends = jnp.cumsum(lengths.astype(jnp.int32))
    starts = ends - lengths
    out = out_call(embeddings, W1, b1.reshape(1, embed_dim), W2,
                   starts.reshape(1, batch), ends.reshape(1, batch))
    return out
